# Initial kernel scaffold; baseline (speedup 1.0000x reference)
import jax
import jax.numpy as jnp
from jax.experimental import pallas as pl


def kernel(x, edge_index, W0, b0, W1, b1, W2, b2):
    raise NotImplementedError("write your pallas kernel here")


# speedup vs baseline: 6.8569x; 6.8569x over previous
import functools

import jax
import jax.numpy as jnp
from jax import lax
from jax.experimental import pallas as pl
from jax.experimental.pallas import tpu as pltpu
from jax.experimental.pallas import tpu_sc as plsc

N_NODES = 10000
D = 128
N_EDGES = 320000

NC = 1
NS = 16
NW = NC * NS
CK = 128
NCHUNK = N_EDGES // CK
BASE_CHUNKS = NCHUNK // NW
EXTRA = NCHUNK - BASE_CHUNKS * NW
N_PAD = 10240
ROWS_PER_TILE = N_PAD // NS

RB = 1000
GRID = N_NODES // RB


def _sc_mesh():
    return plsc.VectorSubcoreMesh(core_axis_name="c", subcore_axis_name="s",
                                  num_cores=NC)


def _deg(dst):
    @functools.partial(
        pl.kernel,
        mesh=_sc_mesh(),
        out_type=jax.ShapeDtypeStruct((NC, N_PAD, D), jnp.float32),
        scratch_types=[
            pltpu.VMEM((CK,), jnp.int32),
            pltpu.VMEM((CK, D), jnp.float32),
            pltpu.VMEM_SHARED((N_PAD, D), jnp.float32),
        ],
    )
    def deg_kernel(dst_hbm, out_hbm, idx_v, ones_v, acc):
        c = lax.axis_index("c")
        s = lax.axis_index("s")
        w = c * NS + s
        zero16 = jnp.zeros((16,), jnp.float32)
        one16 = jnp.ones((16,), jnp.float32)

        def zrow(i, carry):
            for j in range(D // 16):
                ones_v[i, pl.ds(j * 16, 16)] = zero16
            return carry

        lax.fori_loop(0, CK, zrow, 0)

        row0 = s * ROWS_PER_TILE
        for k in range(ROWS_PER_TILE // CK):
            pltpu.sync_copy(ones_v, acc.at[pl.ds(row0 + k * CK, CK)])

        def orow(i, carry):
            for j in range(D // 16):
                ones_v[i, pl.ds(j * 16, 16)] = one16
            return carry

        lax.fori_loop(0, CK, orow, 0)
        plsc.subcore_barrier()

        nchunks = BASE_CHUNKS + jnp.where(w < EXTRA, 1, 0)

        def body(g, carry):
            base = pl.multiple_of((w + g * NW) * CK, CK)
            pltpu.sync_copy(dst_hbm.at[pl.ds(base, CK)], idx_v)
            pltpu.sync_copy(ones_v, acc.at[idx_v], add=True)
            return carry

        lax.fori_loop(0, nchunks, body, 0)

        plsc.subcore_barrier()
        pltpu.sync_copy(
            acc.at[pl.ds(row0, ROWS_PER_TILE)],
            out_hbm.at[c, pl.ds(row0, ROWS_PER_TILE)],
        )

    return deg_kernel(dst)


def _msg(xs, src, dst):
    @functools.partial(
        pl.kernel,
        mesh=_sc_mesh(),
        out_type=jax.ShapeDtypeStruct((NC, N_PAD, D), jnp.float32),
        scratch_types=[
            pltpu.VMEM((CK,), jnp.int32),
            pltpu.VMEM((CK,), jnp.int32),
            pltpu.VMEM((CK, D), jnp.float32),
            pltpu.VMEM_SHARED((N_PAD, D), jnp.float32),
            pltpu.SemaphoreType.DMA,
        ],
    )
    def msg_kernel(xs_hbm, src_hbm, dst_hbm, out_hbm,
                   sidx, didx, rows_v, acc, sem):
        c = lax.axis_index("c")
        s = lax.axis_index("s")
        w = c * NS + s
        zero16 = jnp.zeros((16,), jnp.float32)

        def zrow(i, carry):
            for j in range(D // 16):
                rows_v[i, pl.ds(j * 16, 16)] = zero16
            return carry

        lax.fori_loop(0, CK, zrow, 0)

        row0 = s * ROWS_PER_TILE
        for k in range(ROWS_PER_TILE // CK):
            pltpu.sync_copy(rows_v, acc.at[pl.ds(row0 + k * CK, CK)])
        plsc.subcore_barrier()

        nchunks = BASE_CHUNKS + jnp.where(w < EXTRA, 1, 0)

        def body(g, carry):
            base = pl.multiple_of((w + g * NW) * CK, CK)
            pltpu.sync_copy(src_hbm.at[pl.ds(base, CK)], sidx)
            pltpu.sync_copy(dst_hbm.at[pl.ds(base, CK)], didx)
            pltpu.async_copy(xs_hbm.at[sidx], rows_v, sem).wait()
            pltpu.sync_copy(rows_v, acc.at[didx], add=True)
            return carry

        lax.fori_loop(0, nchunks, body, 0)

        plsc.subcore_barrier()
        pltpu.sync_copy(
            acc.at[pl.ds(row0, ROWS_PER_TILE)],
            out_hbm.at[c, pl.ds(row0, ROWS_PER_TILE)],
        )

    return msg_kernel(xs, src, dst)


def _scales(degp_ref):
    deg = degp_ref[0, :, 0:1] + 1.0
    dis = lax.rsqrt(deg)
    return dis, 1.0 / deg


def _first_body(x_ref, degp_ref, w_ref, b_ref, xs_ref, r_ref):
    dis, inv = _scales(degp_ref)
    x = x_ref[...]
    h = jnp.dot(x, w_ref[...], preferred_element_type=jnp.float32)
    xs_ref[...] = dis * h
    r_ref[...] = inv * h + b_ref[...] + x


def _mid_body(p_ref, rin_ref, degp_ref, w_ref, b_ref, xs_ref, r_ref):
    dis, inv = _scales(degp_ref)
    xn = dis * p_ref[0] + rin_ref[...]
    h = jnp.dot(xn, w_ref[...], preferred_element_type=jnp.float32)
    xs_ref[...] = dis * h
    r_ref[...] = inv * h + b_ref[...] + xn


def _last_body(p_ref, rin_ref, degp_ref, o_ref):
    dis, _ = _scales(degp_ref)
    o_ref[...] = dis * p_ref[0] + rin_ref[...]


_ROWS = pl.BlockSpec((RB, D), lambda i: (i, 0))
_DEGS = pl.BlockSpec((NC, RB, D), lambda i: (0, i, 0))
_PART = pl.BlockSpec((NC, RB, D), lambda i: (0, i, 0))
_WSPEC = pl.BlockSpec((D, D), lambda i: (0, 0))
_BSPEC = pl.BlockSpec((1, D), lambda i: (0, 0))
_XSD = jax.ShapeDtypeStruct((N_NODES, D), jnp.float32)


def _tc_first(x, degp, W, b):
    return pl.pallas_call(
        _first_body,
        grid=(GRID,),
        in_specs=[_ROWS, _DEGS, _WSPEC, _BSPEC],
        out_specs=[_ROWS, _ROWS],
        out_shape=[_XSD, _XSD],
    )(x, degp, W, b)


def _tc_mid(p, rin, degp, W, b):
    return pl.pallas_call(
        _mid_body,
        grid=(GRID,),
        in_specs=[_PART, _ROWS, _DEGS, _WSPEC, _BSPEC],
        out_specs=[_ROWS, _ROWS],
        out_shape=[_XSD, _XSD],
    )(p, rin, degp, W, b)


def _tc_last(p, rin, degp):
    return pl.pallas_call(
        _last_body,
        grid=(GRID,),
        in_specs=[_PART, _ROWS, _DEGS],
        out_specs=_ROWS,
        out_shape=_XSD,
    )(p, rin, degp)


def kernel(x, edge_index, W0, b0, W1, b1, W2, b2):
    src = edge_index[0].astype(jnp.int32)
    dst = edge_index[1].astype(jnp.int32)
    degp = _deg(dst)
    b0r, b1r, b2r = (b.reshape(1, D) for b in (b0, b1, b2))
    xs, r = _tc_first(x, degp, W0, b0r)
    for (W, b) in ((W1, b1r), (W2, b2r)):
        p = _msg(xs, src, dst)
        xs, r = _tc_mid(p, r, degp, W, b)
    p = _msg(xs, src, dst)
    return _tc_last(p, r, degp)


# speedup vs baseline: 14.0390x; 2.0474x over previous
import functools

import jax
import jax.numpy as jnp
from jax import lax
from jax.experimental import pallas as pl
from jax.experimental.pallas import tpu as pltpu
from jax.experimental.pallas import tpu_sc as plsc

N_NODES = 10000
D = 128
N_EDGES = 320000

NC = 1
NS = 16
NW = NC * NS
CK = 128
NCHUNK = N_EDGES // CK
CB = 32
NBATCH = 5
NCHUNK_PAD = 2560
N_PAD = 10240
ROWS_PER_TILE = N_PAD // NS

RB = 1000
GRID = N_NODES // RB


def _sc_mesh():
    return plsc.VectorSubcoreMesh(core_axis_name="c", subcore_axis_name="s",
                                  num_cores=NC)


def _tile_range(w):
    start = jnp.where(w < 8, 160 * w,
                      jnp.where(w < 15, 1280 + 152 * (w - 8), 2344))
    count = jnp.where(w < 8, 160, jnp.where(w < 15, 152, 156))
    return start, count


def _zero_vmem_rows(buf):
    zero16 = jnp.zeros((16,), jnp.float32)

    def zrow(i, carry):
        for j in range(D // 16):
            buf[i, pl.ds(j * 16, 16)] = zero16
        return carry

    lax.fori_loop(0, CK, zrow, 0)


def _deg(dst2):
    @functools.partial(
        pl.kernel,
        mesh=_sc_mesh(),
        out_type=jax.ShapeDtypeStruct((NC, N_PAD, D), jnp.float32),
        scratch_types=[
            pltpu.VMEM((CB, CK), jnp.int32),
            pltpu.VMEM((CK, D), jnp.float32),
            pltpu.VMEM_SHARED((N_PAD, D), jnp.float32),
            pltpu.SemaphoreType.DMA,
        ],
    )
    def deg_kernel(dst_hbm, out_hbm, dbatch, ones_v, acc, sem):
        c = lax.axis_index("c")
        s = lax.axis_index("s")
        w = c * NS + s
        one16 = jnp.ones((16,), jnp.float32)

        _zero_vmem_rows(ones_v)
        row0 = s * ROWS_PER_TILE
        for k in range(ROWS_PER_TILE // CK):
            pltpu.sync_copy(ones_v, acc.at[pl.ds(row0 + k * CK, CK)])

        def orow(i, carry):
            for j in range(D // 16):
                ones_v[i, pl.ds(j * 16, 16)] = one16
            return carry

        lax.fori_loop(0, CK, orow, 0)
        plsc.subcore_barrier()

        start, count = _tile_range(w)

        def batch_body(B, carry):
            t0 = B * CB
            pltpu.sync_copy(dst_hbm.at[pl.ds(start + t0, CB)], dbatch)
            for j in range(CB):
                @pl.when(t0 + j < count)
                def _(j=j):
                    pltpu.async_copy(ones_v, acc.at[dbatch.at[j]], sem,
                                     add=True)
            for j in range(CB):
                @pl.when(t0 + j < count)
                def _(j=j):
                    pltpu.make_async_copy(
                        ones_v, acc.at[dbatch.at[j]], sem).wait()
            return carry

        lax.fori_loop(0, NBATCH, batch_body, 0)

        plsc.subcore_barrier()
        pltpu.sync_copy(
            acc.at[pl.ds(row0, ROWS_PER_TILE)],
            out_hbm.at[c, pl.ds(row0, ROWS_PER_TILE)],
        )

    return deg_kernel(dst2)


def _msg(xs, src2, dst2):
    @functools.partial(
        pl.kernel,
        mesh=_sc_mesh(),
        out_type=jax.ShapeDtypeStruct((NC, N_PAD, D), jnp.float32),
        scratch_types=[
            pltpu.VMEM((CB, CK), jnp.int32),
            pltpu.VMEM((CB, CK), jnp.int32),
            pltpu.VMEM((CK, D), jnp.float32),
            pltpu.VMEM((CK, D), jnp.float32),
            pltpu.VMEM_SHARED((N_PAD, D), jnp.float32),
            pltpu.SemaphoreType.DMA,
            pltpu.SemaphoreType.DMA,
        ],
    )
    def msg_kernel(xs_hbm, src_hbm, dst_hbm, out_hbm,
                   sbatch, dbatch, rows0, rows1, acc, sem0, sem1):
        c = lax.axis_index("c")
        s = lax.axis_index("s")
        w = c * NS + s
        rows = (rows0, rows1)
        sems = (sem0, sem1)

        _zero_vmem_rows(rows0)
        row0 = s * ROWS_PER_TILE
        for k in range(ROWS_PER_TILE // CK):
            pltpu.sync_copy(rows0, acc.at[pl.ds(row0 + k * CK, CK)])
        plsc.subcore_barrier()

        start, count = _tile_range(w)

        def batch_body(B, carry):
            t0 = B * CB
            pltpu.sync_copy(src_hbm.at[pl.ds(start + t0, CB)], sbatch)
            pltpu.sync_copy(dst_hbm.at[pl.ds(start + t0, CB)], dbatch)

            @pl.when(t0 < count)
            def _():
                pltpu.async_copy(xs_hbm.at[sbatch.at[0]], rows0, sem0)

            for j in range(CB):
                b = j % 2

                @pl.when(t0 + j < count)
                def _(j=j, b=b):
                    if j + 1 < CB:
                        @pl.when(t0 + j + 1 < count)
                        def _():
                            pltpu.async_copy(
                                xs_hbm.at[sbatch.at[j + 1]],
                                rows[(j + 1) % 2], sems[(j + 1) % 2])
                    pltpu.make_async_copy(
                        xs_hbm.at[sbatch.at[j]], rows[b], sems[b]).wait()
                    pltpu.sync_copy(rows[b], acc.at[dbatch.at[j]], add=True)
            return carry

        lax.fori_loop(0, NBATCH, batch_body, 0)

        plsc.subcore_barrier()
        pltpu.sync_copy(
            acc.at[pl.ds(row0, ROWS_PER_TILE)],
            out_hbm.at[c, pl.ds(row0, ROWS_PER_TILE)],
        )

    return msg_kernel(xs, src2, dst2)


def _scales(degp_ref):
    deg = degp_ref[0, :, 0:1] + 1.0
    dis = lax.rsqrt(deg)
    return dis, 1.0 / deg


def _first_body(x_ref, degp_ref, w_ref, b_ref, xs_ref, r_ref):
    dis, inv = _scales(degp_ref)
    x = x_ref[...]
    h = jnp.dot(x, w_ref[...], preferred_element_type=jnp.float32)
    xs_ref[...] = dis * h
    r_ref[...] = inv * h + b_ref[...] + x


def _mid_body(p_ref, rin_ref, degp_ref, w_ref, b_ref, xs_ref, r_ref):
    dis, inv = _scales(degp_ref)
    xn = dis * p_ref[0] + rin_ref[...]
    h = jnp.dot(xn, w_ref[...], preferred_element_type=jnp.float32)
    xs_ref[...] = dis * h
    r_ref[...] = inv * h + b_ref[...] + xn


def _last_body(p_ref, rin_ref, degp_ref, o_ref):
    dis, _ = _scales(degp_ref)
    o_ref[...] = dis * p_ref[0] + rin_ref[...]


_ROWS = pl.BlockSpec((RB, D), lambda i: (i, 0))
_DEGS = pl.BlockSpec((NC, RB, D), lambda i: (0, i, 0))
_PART = pl.BlockSpec((NC, RB, D), lambda i: (0, i, 0))
_WSPEC = pl.BlockSpec((D, D), lambda i: (0, 0))
_BSPEC = pl.BlockSpec((1, D), lambda i: (0, 0))
_XSD = jax.ShapeDtypeStruct((N_NODES, D), jnp.float32)


def _tc_first(x, degp, W, b):
    return pl.pallas_call(
        _first_body,
        grid=(GRID,),
        in_specs=[_ROWS, _DEGS, _WSPEC, _BSPEC],
        out_specs=[_ROWS, _ROWS],
        out_shape=[_XSD, _XSD],
    )(x, degp, W, b)


def _tc_mid(p, rin, degp, W, b):
    return pl.pallas_call(
        _mid_body,
        grid=(GRID,),
        in_specs=[_PART, _ROWS, _DEGS, _WSPEC, _BSPEC],
        out_specs=[_ROWS, _ROWS],
        out_shape=[_XSD, _XSD],
    )(p, rin, degp, W, b)


def _tc_last(p, rin, degp):
    return pl.pallas_call(
        _last_body,
        grid=(GRID,),
        in_specs=[_PART, _ROWS, _DEGS],
        out_specs=_ROWS,
        out_shape=_XSD,
    )(p, rin, degp)


def kernel(x, edge_index, W0, b0, W1, b1, W2, b2):
    src = edge_index[0].astype(jnp.int32)
    dst = edge_index[1].astype(jnp.int32)
    pad = NCHUNK_PAD * CK - N_EDGES
    src2 = jnp.pad(src, (0, pad)).reshape(NCHUNK_PAD, CK)
    dst2 = jnp.pad(dst, (0, pad)).reshape(NCHUNK_PAD, CK)
    degp = _deg(dst2)
    b0r, b1r, b2r = (b.reshape(1, D) for b in (b0, b1, b2))
    xs, r = _tc_first(x, degp, W0, b0r)
    for (W, b) in ((W1, b1r), (W2, b2r)):
        p = _msg(xs, src2, dst2)
        xs, r = _tc_mid(p, r, degp, W, b)
    p = _msg(xs, src2, dst2)
    return _tc_last(p, r, degp)


# speedup vs baseline: 23.0456x; 1.6415x over previous
import functools

import jax
import jax.numpy as jnp
from jax import lax
from jax.experimental import pallas as pl
from jax.experimental.pallas import tpu as pltpu
from jax.experimental.pallas import tpu_sc as plsc

N_NODES = 10000
D = 128
N_EDGES = 320000

NC = 2
NS = 16
NW = NC * NS
CK = 128
NCHUNK = N_EDGES // CK
CB = 32
NBATCH = 3
NCHUNK_PAD = 2560
N_PAD = 10240
ROWS_PER_TILE = N_PAD // NS

RB = 1000
GRID = N_NODES // RB


def _sc_mesh():
    return plsc.VectorSubcoreMesh(core_axis_name="c", subcore_axis_name="s",
                                  num_cores=NC)


def _tile_range(w):
    start = jnp.where(w < 24, 80 * w,
                      jnp.where(w < 31, 1920 + 72 * (w - 24), 2424))
    count = jnp.where(w < 24, 80, jnp.where(w < 31, 72, 76))
    return start, count


def _zero_vmem_rows(buf):
    zero16 = jnp.zeros((16,), jnp.float32)

    def zrow(i, carry):
        for j in range(D // 16):
            buf[i, pl.ds(j * 16, 16)] = zero16
        return carry

    lax.fori_loop(0, CK, zrow, 0)


def _deg(dst2):
    @functools.partial(
        pl.kernel,
        mesh=_sc_mesh(),
        out_type=jax.ShapeDtypeStruct((NC, N_PAD, D), jnp.float32),
        scratch_types=[
            pltpu.VMEM((CB, CK), jnp.int32),
            pltpu.VMEM((CK, D), jnp.float32),
            pltpu.VMEM_SHARED((N_PAD, D), jnp.float32),
            pltpu.SemaphoreType.DMA,
        ],
    )
    def deg_kernel(dst_hbm, out_hbm, dbatch, ones_v, acc, sem):
        c = lax.axis_index("c")
        s = lax.axis_index("s")
        w = c * NS + s
        one16 = jnp.ones((16,), jnp.float32)

        _zero_vmem_rows(ones_v)
        row0 = s * ROWS_PER_TILE
        for k in range(ROWS_PER_TILE // CK):
            pltpu.sync_copy(ones_v, acc.at[pl.ds(row0 + k * CK, CK)])

        def orow(i, carry):
            for j in range(D // 16):
                ones_v[i, pl.ds(j * 16, 16)] = one16
            return carry

        lax.fori_loop(0, CK, orow, 0)
        plsc.subcore_barrier()

        start, count = _tile_range(w)

        def batch_body(B, carry):
            t0 = B * CB
            pltpu.sync_copy(dst_hbm.at[pl.ds(start + t0, CB)], dbatch)
            for j in range(CB):
                @pl.when(t0 + j < count)
                def _(j=j):
                    pltpu.async_copy(ones_v, acc.at[dbatch.at[j]], sem,
                                     add=True)
            for j in range(CB):
                @pl.when(t0 + j < count)
                def _(j=j):
                    pltpu.make_async_copy(
                        ones_v, acc.at[dbatch.at[j]], sem).wait()
            return carry

        lax.fori_loop(0, NBATCH, batch_body, 0)

        plsc.subcore_barrier()
        pltpu.sync_copy(
            acc.at[pl.ds(row0, ROWS_PER_TILE)],
            out_hbm.at[c, pl.ds(row0, ROWS_PER_TILE)],
        )

    return deg_kernel(dst2)


def _msg(xs, src2, dst2):
    @functools.partial(
        pl.kernel,
        mesh=_sc_mesh(),
        out_type=jax.ShapeDtypeStruct((NC, N_PAD, D), jnp.float32),
        scratch_types=[
            pltpu.VMEM((CB, CK), jnp.int32),
            pltpu.VMEM((CB, CK), jnp.int32),
            pltpu.VMEM((CK, D), jnp.float32),
            pltpu.VMEM((CK, D), jnp.float32),
            pltpu.VMEM_SHARED((N_PAD, D), jnp.float32),
            pltpu.SemaphoreType.DMA,
            pltpu.SemaphoreType.DMA,
        ],
    )
    def msg_kernel(xs_hbm, src_hbm, dst_hbm, out_hbm,
                   sbatch, dbatch, rows0, rows1, acc, sem0, sem1):
        c = lax.axis_index("c")
        s = lax.axis_index("s")
        w = c * NS + s
        rows = (rows0, rows1)
        sems = (sem0, sem1)

        _zero_vmem_rows(rows0)
        row0 = s * ROWS_PER_TILE
        for k in range(ROWS_PER_TILE // CK):
            pltpu.sync_copy(rows0, acc.at[pl.ds(row0 + k * CK, CK)])
        plsc.subcore_barrier()

        start, count = _tile_range(w)

        def batch_body(B, carry):
            t0 = B * CB
            pltpu.sync_copy(src_hbm.at[pl.ds(start + t0, CB)], sbatch)
            pltpu.sync_copy(dst_hbm.at[pl.ds(start + t0, CB)], dbatch)

            @pl.when(t0 < count)
            def _():
                pltpu.async_copy(xs_hbm.at[sbatch.at[0]], rows0, sem0)

            for j in range(CB):
                b = j % 2

                @pl.when(t0 + j < count)
                def _(j=j, b=b):
                    if j + 1 < CB:
                        @pl.when(t0 + j + 1 < count)
                        def _():
                            pltpu.async_copy(
                                xs_hbm.at[sbatch.at[j + 1]],
                                rows[(j + 1) % 2], sems[(j + 1) % 2])
                    pltpu.make_async_copy(
                        xs_hbm.at[sbatch.at[j]], rows[b], sems[b]).wait()
                    pltpu.sync_copy(rows[b], acc.at[dbatch.at[j]], add=True)
            return carry

        lax.fori_loop(0, NBATCH, batch_body, 0)

        plsc.subcore_barrier()
        pltpu.sync_copy(
            acc.at[pl.ds(row0, ROWS_PER_TILE)],
            out_hbm.at[c, pl.ds(row0, ROWS_PER_TILE)],
        )

    return msg_kernel(xs, src2, dst2)


def _scales(degp_ref):
    deg = degp_ref[0, :, 0:1] + degp_ref[1, :, 0:1] + 1.0
    dis = lax.rsqrt(deg)
    return dis, 1.0 / deg


def _psum(p_ref):
    return p_ref[0] + p_ref[1]


def _first_body(x_ref, degp_ref, w_ref, b_ref, xs_ref, r_ref):
    dis, inv = _scales(degp_ref)
    x = x_ref[...]
    h = jnp.dot(x, w_ref[...], preferred_element_type=jnp.float32)
    xs_ref[...] = dis * h
    r_ref[...] = inv * h + b_ref[...] + x


def _mid_body(p_ref, rin_ref, degp_ref, w_ref, b_ref, xs_ref, r_ref):
    dis, inv = _scales(degp_ref)
    xn = dis * _psum(p_ref) + rin_ref[...]
    h = jnp.dot(xn, w_ref[...], preferred_element_type=jnp.float32)
    xs_ref[...] = dis * h
    r_ref[...] = inv * h + b_ref[...] + xn


def _last_body(p_ref, rin_ref, degp_ref, o_ref):
    dis, _ = _scales(degp_ref)
    o_ref[...] = dis * _psum(p_ref) + rin_ref[...]


_ROWS = pl.BlockSpec((RB, D), lambda i: (i, 0))
_DEGS = pl.BlockSpec((NC, RB, D), lambda i: (0, i, 0))
_PART = pl.BlockSpec((NC, RB, D), lambda i: (0, i, 0))
_WSPEC = pl.BlockSpec((D, D), lambda i: (0, 0))
_BSPEC = pl.BlockSpec((1, D), lambda i: (0, 0))
_XSD = jax.ShapeDtypeStruct((N_NODES, D), jnp.float32)


def _tc_first(x, degp, W, b):
    return pl.pallas_call(
        _first_body,
        grid=(GRID,),
        in_specs=[_ROWS, _DEGS, _WSPEC, _BSPEC],
        out_specs=[_ROWS, _ROWS],
        out_shape=[_XSD, _XSD],
    )(x, degp, W, b)


def _tc_mid(p, rin, degp, W, b):
    return pl.pallas_call(
        _mid_body,
        grid=(GRID,),
        in_specs=[_PART, _ROWS, _DEGS, _WSPEC, _BSPEC],
        out_specs=[_ROWS, _ROWS],
        out_shape=[_XSD, _XSD],
    )(p, rin, degp, W, b)


def _tc_last(p, rin, degp):
    return pl.pallas_call(
        _last_body,
        grid=(GRID,),
        in_specs=[_PART, _ROWS, _DEGS],
        out_specs=_ROWS,
        out_shape=_XSD,
    )(p, rin, degp)


def kernel(x, edge_index, W0, b0, W1, b1, W2, b2):
    src = edge_index[0].astype(jnp.int32)
    dst = edge_index[1].astype(jnp.int32)
    pad = NCHUNK_PAD * CK - N_EDGES
    src2 = jnp.pad(src, (0, pad)).reshape(NCHUNK_PAD, CK)
    dst2 = jnp.pad(dst, (0, pad)).reshape(NCHUNK_PAD, CK)
    degp = _deg(dst2)
    b0r, b1r, b2r = (b.reshape(1, D) for b in (b0, b1, b2))
    xs, r = _tc_first(x, degp, W0, b0r)
    for (W, b) in ((W1, b1r), (W2, b2r)):
        p = _msg(xs, src2, dst2)
        xs, r = _tc_mid(p, r, degp, W, b)
    p = _msg(xs, src2, dst2)
    return _tc_last(p, r, degp)
